# Initial kernel scaffold; baseline (speedup 1.0000x reference)
import jax
import jax.numpy as jnp
from jax.experimental import pallas as pl


def kernel(x, edge_index, edge_attr, u, batch, W1, b1, W2, b2):
    raise NotImplementedError("write your pallas kernel here")


# speedup vs baseline: 5.4014x; 5.4014x over previous
import functools

import jax
import jax.numpy as jnp
from jax import lax
from jax.experimental import pallas as pl
from jax.experimental.pallas import tpu as pltpu
from jax.experimental.pallas import tpu_sc as plsc

N_NODES = 10000
N_EDGES = 320000
D = 128
D_EDGE = 16
N_GRAPHS = 16
D_U = 32

NC = 2
NS = 16
NW = NC * NS
C = 80
CHUNKS_PER_TILE = N_EDGES // (NW * C)
NPAD = 10240
ROWS_PER_SUB = NPAD // NS
G = 5
N_GROUPS = CHUNKS_PER_TILE // G


def _tc_pre_body(x_ref, w1_ref, b1_ref, u_ref, batch_ref, td_ref, ts_ref):
    x = x_ref[...]
    w1 = w1_ref[...]
    td_ref[...] = jnp.dot(x, w1[0:D, :], preferred_element_type=jnp.float32)
    u1 = jnp.dot(u_ref[...], w1[2 * D + D_EDGE:, :],
                 preferred_element_type=jnp.float32)
    onehot = (batch_ref[...] == lax.broadcasted_iota(
        jnp.int32, (N_NODES, N_GRAPHS), 1)).astype(jnp.float32)
    ts_ref[...] = (jnp.dot(x, w1[D:2 * D, :], preferred_element_type=jnp.float32)
                   + jnp.dot(onehot, u1, preferred_element_type=jnp.float32)
                   + b1_ref[...])


def _tc_edge_body(ea_ref, w1c_ref, et_ref):
    et_ref[...] = jnp.dot(ea_ref[...], w1c_ref[...],
                          preferred_element_type=jnp.float32)


def _tc_post_body(x_ref, agg2_ref, u_ref, batch_ref, w2_ref, b2_ref, out_ref):
    w2 = w2_ref[...]
    agg = agg2_ref[0, :N_NODES, :] + agg2_ref[1, :N_NODES, :]
    u2 = jnp.dot(u_ref[...], w2[2 * D:, :], preferred_element_type=jnp.float32)
    onehot = (batch_ref[...] == lax.broadcasted_iota(
        jnp.int32, (N_NODES, N_GRAPHS), 1)).astype(jnp.float32)
    acc = (jnp.dot(x_ref[...], w2[0:D, :], preferred_element_type=jnp.float32)
           + jnp.dot(agg, w2[D:2 * D, :], preferred_element_type=jnp.float32)
           + jnp.dot(onehot, u2, preferred_element_type=jnp.float32)
           + b2_ref[...])
    out_ref[...] = jnp.maximum(acc, 0.0)


def _sc_body(td_hbm, ts_hbm, et_hbm, didx_hbm, sidx_hbm, out_hbm,
             didx_v, sidx_v, bufd, bufs, bufe, agg_sh,
             sem_d, sem_s, sem_e):
    c = lax.axis_index("c")
    s = lax.axis_index("s")
    wid = c * NS + s
    row0 = wid * CHUNKS_PER_TILE

    zero16 = jnp.zeros((16,), jnp.float32)

    def _zero_row(i, _):
        for j in range(D // 16):
            bufd[i, pl.ds(j * 16, 16)] = zero16
        return 0

    lax.fori_loop(0, C, _zero_row, 0)
    for k in range(ROWS_PER_SUB // C):
        pltpu.sync_copy(bufd, agg_sh.at[pl.ds(s * ROWS_PER_SUB + k * C, C)])
    plsc.subcore_barrier()

    def _group(g, _):
        pltpu.sync_copy(didx_hbm.at[wid, g], didx_v)
        pltpu.sync_copy(sidx_hbm.at[wid, g], sidx_v)
        for j in range(G):
            di = didx_v.at[j]
            si = sidx_v.at[j]
            cd = pltpu.async_copy(td_hbm.at[di], bufd, sem_d)
            cs = pltpu.async_copy(ts_hbm.at[si], bufs, sem_s)
            ce = pltpu.async_copy(et_hbm.at[row0 + g * G + j], bufe, sem_e)
            cd.wait()
            cs.wait()
            ce.wait()

            def _row(t, _):
                for jj in range(D // 16):
                    sl = pl.ds(jj * 16, 16)
                    bufd[t, sl] = jnp.maximum(
                        bufd[t, sl] + bufs[t, sl] + bufe[t, sl], 0.0)
                return 0

            lax.fori_loop(0, C, _row, 0)
            pltpu.sync_copy(bufd, agg_sh.at[di], add=True)
        return 0

    lax.fori_loop(0, N_GROUPS, _group, 0)
    plsc.subcore_barrier()

    for k in range(ROWS_PER_SUB // C):
        rr = s * ROWS_PER_SUB + k * C
        pltpu.sync_copy(agg_sh.at[pl.ds(rr, C)], out_hbm.at[c, pl.ds(rr, C)])


@functools.cache
def _sc_call():
    return functools.partial(
        pl.kernel,
        out_type=jax.ShapeDtypeStruct((NC, NPAD, D), jnp.float32),
        mesh=plsc.VectorSubcoreMesh(core_axis_name="c", subcore_axis_name="s",
                                    num_cores=NC, num_subcores=NS),
        scratch_types=[
            pltpu.VMEM((G, C), jnp.int32),
            pltpu.VMEM((G, C), jnp.int32),
            pltpu.VMEM((C, D), jnp.float32),
            pltpu.VMEM((C, D), jnp.float32),
            pltpu.VMEM((C, D), jnp.float32),
            pltpu.VMEM_SHARED((NPAD, D), jnp.float32),
            pltpu.SemaphoreType.DMA,
            pltpu.SemaphoreType.DMA,
            pltpu.SemaphoreType.DMA,
        ],
    )(_sc_body)


def kernel(x, edge_index, edge_attr, u, batch, W1, b1, W2, b2):
    src = edge_index[0].astype(jnp.int32)
    dst = edge_index[1].astype(jnp.int32)
    batch2d = batch.astype(jnp.int32).reshape(N_NODES, 1)

    td, ts = pl.pallas_call(
        _tc_pre_body,
        out_shape=[jax.ShapeDtypeStruct((N_NODES, D), jnp.float32)] * 2,
    )(x, W1, b1.reshape(1, D), u, batch2d)

    BE = 16000
    eterm = pl.pallas_call(
        _tc_edge_body,
        grid=(N_EDGES // BE,),
        in_specs=[
            pl.BlockSpec((BE, D_EDGE), lambda i: (i, 0)),
            pl.BlockSpec((D_EDGE, D), lambda i: (0, 0)),
        ],
        out_specs=pl.BlockSpec((BE, D), lambda i: (i, 0)),
        out_shape=jax.ShapeDtypeStruct((N_EDGES, D), jnp.float32),
    )(edge_attr, W1[2 * D:2 * D + D_EDGE, :])

    agg2 = _sc_call()(td, ts,
                      eterm.reshape(N_EDGES // C, C, D),
                      dst.reshape(NW, N_GROUPS, G, C),
                      src.reshape(NW, N_GROUPS, G, C))

    out = pl.pallas_call(
        _tc_post_body,
        out_shape=jax.ShapeDtypeStruct((N_NODES, D), jnp.float32),
    )(x, agg2, u, batch2d, W2, b2.reshape(1, D))
    return out


# speedup vs baseline: 6.6479x; 1.2308x over previous
import functools

import jax
import jax.numpy as jnp
from jax import lax
from jax.experimental import pallas as pl
from jax.experimental.pallas import tpu as pltpu
from jax.experimental.pallas import tpu_sc as plsc

N_NODES = 10000
N_EDGES = 320000
D = 128
D_EDGE = 16
N_GRAPHS = 16
D_U = 32

NC = 2
NS = 16
NW = NC * NS
C = 40
CHUNKS_PER_TILE = N_EDGES // (NW * C)
NPAD = 10240
ROWS_PER_SUB = NPAD // NS
G = 10
N_GROUPS = CHUNKS_PER_TILE // G


def _tc_pre_body(x_ref, w1_ref, b1_ref, u_ref, batch_ref, td_ref, ts_ref):
    x = x_ref[...]
    w1 = w1_ref[...]
    td_ref[...] = jnp.dot(x, w1[0:D, :], preferred_element_type=jnp.float32)
    u1 = jnp.dot(u_ref[...], w1[2 * D + D_EDGE:, :],
                 preferred_element_type=jnp.float32)
    onehot = (batch_ref[...] == lax.broadcasted_iota(
        jnp.int32, (N_NODES, N_GRAPHS), 1)).astype(jnp.float32)
    ts_ref[...] = (jnp.dot(x, w1[D:2 * D, :], preferred_element_type=jnp.float32)
                   + jnp.dot(onehot, u1, preferred_element_type=jnp.float32)
                   + b1_ref[...])


def _tc_edge_body(ea_ref, w1c_ref, et_ref):
    be = ea_ref.shape[0]
    et_ref[...] = jnp.dot(ea_ref[...], w1c_ref[...],
                          preferred_element_type=jnp.float32
                          ).reshape(be // C, C, D)


def _tc_post_body(x_ref, agg2_ref, u_ref, batch_ref, w2_ref, b2_ref, out_ref):
    w2 = w2_ref[...]
    agg = agg2_ref[0, :N_NODES, :] + agg2_ref[1, :N_NODES, :]
    u2 = jnp.dot(u_ref[...], w2[2 * D:, :], preferred_element_type=jnp.float32)
    onehot = (batch_ref[...] == lax.broadcasted_iota(
        jnp.int32, (N_NODES, N_GRAPHS), 1)).astype(jnp.float32)
    acc = (jnp.dot(x_ref[...], w2[0:D, :], preferred_element_type=jnp.float32)
           + jnp.dot(agg, w2[D:2 * D, :], preferred_element_type=jnp.float32)
           + jnp.dot(onehot, u2, preferred_element_type=jnp.float32)
           + b2_ref[...])
    out_ref[...] = jnp.maximum(acc, 0.0)


def _sc_body(td_hbm, ts_hbm, et_hbm, didx_hbm, sidx_hbm, out_hbm,
             didx_v, sidx_v, bufd0, bufs0, bufe0, bufd1, bufs1, bufe1, agg_sh,
             sem_d0, sem_s0, sem_e0, sem_d1, sem_s1, sem_e1, sem_i):
    c = lax.axis_index("c")
    s = lax.axis_index("s")
    wid = c * NS + s
    row0 = wid * CHUNKS_PER_TILE
    sets = ((bufd0, bufs0, bufe0, sem_d0, sem_s0, sem_e0),
            (bufd1, bufs1, bufe1, sem_d1, sem_s1, sem_e1))

    zero16 = jnp.zeros((16,), jnp.float32)

    def _zero_row(i, _):
        for j in range(D // 16):
            bufd0[i, pl.ds(j * 16, 16)] = zero16
        return 0

    lax.fori_loop(0, C, _zero_row, 0)
    for k in range(ROWS_PER_SUB // C):
        pltpu.sync_copy(bufd0, agg_sh.at[pl.ds(s * ROWS_PER_SUB + k * C, C)])
    plsc.subcore_barrier()

    def _issue(set_idx_refs, chunk):
        bd, bs, be, sd, ss, se = set_idx_refs[0]
        di, si = set_idx_refs[1]
        pltpu.async_copy(td_hbm.at[di], bd, sd)
        pltpu.async_copy(ts_hbm.at[si], bs, ss)
        pltpu.async_copy(et_hbm.at[chunk], be, se)

    pltpu.sync_copy(didx_hbm.at[wid, 0], didx_v.at[0])
    pltpu.sync_copy(sidx_hbm.at[wid, 0], sidx_v.at[0])
    _issue((sets[0], (didx_v.at[0, 0], sidx_v.at[0, 0])), row0)

    def _group(g, _):
        p = lax.rem(g, 2)
        np_ = 1 - p
        nxt = jnp.minimum(g + 1, N_GROUPS - 1)
        ci1 = pltpu.async_copy(didx_hbm.at[wid, nxt], didx_v.at[np_], sem_i)
        ci2 = pltpu.async_copy(sidx_hbm.at[wid, nxt], sidx_v.at[np_], sem_i)
        base = row0 + g * G
        for j in range(G):
            bd, bs, be, sd, ss, se = sets[j % 2]
            di = didx_v.at[p, j]
            pltpu.make_async_copy(td_hbm.at[di], bd, sd).wait()
            pltpu.make_async_copy(td_hbm.at[di], bs, ss).wait()
            pltpu.make_async_copy(et_hbm.at[base + j], be, se).wait()
            if j < G - 1:
                _issue((sets[(j + 1) % 2],
                        (didx_v.at[p, j + 1], sidx_v.at[p, j + 1])),
                       base + j + 1)
            else:
                ci1.wait()
                ci2.wait()

                @pl.when(g < N_GROUPS - 1)
                def _():
                    _issue((sets[0],
                            (didx_v.at[np_, 0], sidx_v.at[np_, 0])),
                           base + G)

            def _row(t, _):
                for jj in range(D // 16):
                    sl = pl.ds(jj * 16, 16)
                    bd[t, sl] = jnp.maximum(
                        bd[t, sl] + bs[t, sl] + be[t, sl], 0.0)
                return 0

            lax.fori_loop(0, C, _row, 0)
            pltpu.sync_copy(bd, agg_sh.at[di], add=True)
        return 0

    lax.fori_loop(0, N_GROUPS, _group, 0)
    plsc.subcore_barrier()

    for k in range(ROWS_PER_SUB // C):
        rr = s * ROWS_PER_SUB + k * C
        pltpu.sync_copy(agg_sh.at[pl.ds(rr, C)], out_hbm.at[c, pl.ds(rr, C)])


@functools.cache
def _sc_call():
    return functools.partial(
        pl.kernel,
        out_type=jax.ShapeDtypeStruct((NC, NPAD, D), jnp.float32),
        mesh=plsc.VectorSubcoreMesh(core_axis_name="c", subcore_axis_name="s",
                                    num_cores=NC, num_subcores=NS),
        scratch_types=[
            pltpu.VMEM((2, G, C), jnp.int32),
            pltpu.VMEM((2, G, C), jnp.int32),
            pltpu.VMEM((C, D), jnp.float32),
            pltpu.VMEM((C, D), jnp.float32),
            pltpu.VMEM((C, D), jnp.float32),
            pltpu.VMEM((C, D), jnp.float32),
            pltpu.VMEM((C, D), jnp.float32),
            pltpu.VMEM((C, D), jnp.float32),
            pltpu.VMEM_SHARED((NPAD, D), jnp.float32),
            pltpu.SemaphoreType.DMA,
            pltpu.SemaphoreType.DMA,
            pltpu.SemaphoreType.DMA,
            pltpu.SemaphoreType.DMA,
            pltpu.SemaphoreType.DMA,
            pltpu.SemaphoreType.DMA,
            pltpu.SemaphoreType.DMA,
        ],
    )(_sc_body)


def kernel(x, edge_index, edge_attr, u, batch, W1, b1, W2, b2):
    src = edge_index[0].astype(jnp.int32)
    dst = edge_index[1].astype(jnp.int32)
    batch2d = batch.astype(jnp.int32).reshape(N_NODES, 1)

    td, ts = pl.pallas_call(
        _tc_pre_body,
        out_shape=[jax.ShapeDtypeStruct((N_NODES, D), jnp.float32)] * 2,
    )(x, W1, b1.reshape(1, D), u, batch2d)

    BE = 16000
    eterm = pl.pallas_call(
        _tc_edge_body,
        grid=(N_EDGES // BE,),
        in_specs=[
            pl.BlockSpec((BE, D_EDGE), lambda i: (i, 0)),
            pl.BlockSpec((D_EDGE, D), lambda i: (0, 0)),
        ],
        out_specs=pl.BlockSpec((BE // C, C, D), lambda i: (i, 0, 0)),
        out_shape=jax.ShapeDtypeStruct((N_EDGES // C, C, D), jnp.float32),
    )(edge_attr, W1[2 * D:2 * D + D_EDGE, :])

    agg2 = _sc_call()(td, ts, eterm,
                      dst.reshape(NW, N_GROUPS, G, C),
                      src.reshape(NW, N_GROUPS, G, C))

    out = pl.pallas_call(
        _tc_post_body,
        out_shape=jax.ShapeDtypeStruct((N_NODES, D), jnp.float32),
    )(x, agg2, u, batch2d, W2, b2.reshape(1, D))
    return out


# speedup vs baseline: 6.6850x; 1.0056x over previous
import functools

import jax
import jax.numpy as jnp
from jax import lax
from jax.experimental import pallas as pl
from jax.experimental.pallas import tpu as pltpu
from jax.experimental.pallas import tpu_sc as plsc

N_NODES = 10000
N_EDGES = 320000
D = 128
D_EDGE = 16
N_GRAPHS = 16
D_U = 32
IN1 = 2 * D + D_EDGE + D_U

NC = 2
NS = 16
NW = NC * NS
C = 40
CHUNKS_PER_TILE = N_EDGES // (NW * C)
NPAD = 10240
ROWS_PER_SUB = NPAD // NS
G = 10
N_GROUPS = CHUNKS_PER_TILE // G


def _tc_pre_body(x_ref, w1_ref, b1_ref, u_ref, batch_ref, ea_ref,
                 td_ref, ts_ref, et_ref):
    be = ea_ref.shape[0]
    w1 = w1_ref[...]
    et_ref[...] = jnp.dot(ea_ref[...], w1[2 * D:2 * D + D_EDGE, :],
                          preferred_element_type=jnp.float32
                          ).reshape(be // C, C, D)

    @pl.when(pl.program_id(0) == 0)
    def _():
        x = x_ref[...]
        td_ref[...] = jnp.dot(x, w1[0:D, :], preferred_element_type=jnp.float32)
        u1 = jnp.dot(u_ref[...], w1[2 * D + D_EDGE:, :],
                     preferred_element_type=jnp.float32)
        onehot = (batch_ref[...] == lax.broadcasted_iota(
            jnp.int32, (N_NODES, N_GRAPHS), 1)).astype(jnp.float32)
        ts_ref[...] = (jnp.dot(x, w1[D:2 * D, :],
                               preferred_element_type=jnp.float32)
                       + jnp.dot(onehot, u1, preferred_element_type=jnp.float32)
                       + b1_ref[...])


def _tc_post_body(x_ref, agg2_ref, u_ref, batch_ref, w2_ref, b2_ref, out_ref):
    w2 = w2_ref[...]
    agg = agg2_ref[0, :N_NODES, :] + agg2_ref[1, :N_NODES, :]
    u2 = jnp.dot(u_ref[...], w2[2 * D:, :], preferred_element_type=jnp.float32)
    onehot = (batch_ref[...] == lax.broadcasted_iota(
        jnp.int32, (N_NODES, N_GRAPHS), 1)).astype(jnp.float32)
    acc = (jnp.dot(x_ref[...], w2[0:D, :], preferred_element_type=jnp.float32)
           + jnp.dot(agg, w2[D:2 * D, :], preferred_element_type=jnp.float32)
           + jnp.dot(onehot, u2, preferred_element_type=jnp.float32)
           + b2_ref[...])
    out_ref[...] = jnp.maximum(acc, 0.0)


def _sc_body(td_hbm, ts_hbm, et_hbm, didx_hbm, sidx_hbm, out_hbm,
             didx_v, sidx_v, bufd0, bufs0, bufe0, bufd1, bufs1, bufe1, agg_sh,
             sem_d0, sem_s0, sem_e0, sem_d1, sem_s1, sem_e1, sem_i):
    c = lax.axis_index("c")
    s = lax.axis_index("s")
    wid = c * NS + s
    row0 = wid * CHUNKS_PER_TILE
    sets = ((bufd0, bufs0, bufe0, sem_d0, sem_s0, sem_e0),
            (bufd1, bufs1, bufe1, sem_d1, sem_s1, sem_e1))

    zero16 = jnp.zeros((16,), jnp.float32)

    def _zero_row(i, _):
        for j in range(D // 16):
            bufd0[i, pl.ds(j * 16, 16)] = zero16
        return 0

    lax.fori_loop(0, C, _zero_row, 0)
    for k in range(ROWS_PER_SUB // C):
        pltpu.sync_copy(bufd0, agg_sh.at[pl.ds(s * ROWS_PER_SUB + k * C, C)])
    plsc.subcore_barrier()

    def _issue(set_idx_refs, chunk):
        bd, bs, be, sd, ss, se = set_idx_refs[0]
        di, si = set_idx_refs[1]
        pltpu.async_copy(td_hbm.at[di], bd, sd)
        pltpu.async_copy(ts_hbm.at[si], bs, ss)
        pltpu.async_copy(et_hbm.at[chunk], be, se)

    pltpu.sync_copy(didx_hbm.at[wid, 0], didx_v.at[0])
    pltpu.sync_copy(sidx_hbm.at[wid, 0], sidx_v.at[0])
    _issue((sets[0], (didx_v.at[0, 0], sidx_v.at[0, 0])), row0)

    def _group(g, _):
        p = lax.rem(g, 2)
        np_ = 1 - p
        nxt = jnp.minimum(g + 1, N_GROUPS - 1)
        ci1 = pltpu.async_copy(didx_hbm.at[wid, nxt], didx_v.at[np_], sem_i)
        ci2 = pltpu.async_copy(sidx_hbm.at[wid, nxt], sidx_v.at[np_], sem_i)
        base = row0 + g * G
        for j in range(G):
            bd, bs, be, sd, ss, se = sets[j % 2]
            di = didx_v.at[p, j]
            pltpu.make_async_copy(td_hbm.at[di], bd, sd).wait()
            pltpu.make_async_copy(td_hbm.at[di], bs, ss).wait()
            pltpu.make_async_copy(et_hbm.at[base + j], be, se).wait()
            if j < G - 1:
                _issue((sets[(j + 1) % 2],
                        (didx_v.at[p, j + 1], sidx_v.at[p, j + 1])),
                       base + j + 1)
            else:
                ci1.wait()
                ci2.wait()

                @pl.when(g < N_GROUPS - 1)
                def _():
                    _issue((sets[0],
                            (didx_v.at[np_, 0], sidx_v.at[np_, 0])),
                           base + G)

            def _row(t, _):
                for jj in range(D // 16):
                    sl = pl.ds(jj * 16, 16)
                    bd[t, sl] = jnp.maximum(
                        bd[t, sl] + bs[t, sl] + be[t, sl], 0.0)
                return 0

            lax.fori_loop(0, C, _row, 0)
            pltpu.sync_copy(bd, agg_sh.at[di], add=True)
        return 0

    lax.fori_loop(0, N_GROUPS, _group, 0)
    plsc.subcore_barrier()

    for k in range(ROWS_PER_SUB // C):
        rr = s * ROWS_PER_SUB + k * C
        pltpu.sync_copy(agg_sh.at[pl.ds(rr, C)], out_hbm.at[c, pl.ds(rr, C)])


@functools.cache
def _sc_call():
    return functools.partial(
        pl.kernel,
        out_type=jax.ShapeDtypeStruct((NC, NPAD, D), jnp.float32),
        mesh=plsc.VectorSubcoreMesh(core_axis_name="c", subcore_axis_name="s",
                                    num_cores=NC, num_subcores=NS),
        scratch_types=[
            pltpu.VMEM((2, G, C), jnp.int32),
            pltpu.VMEM((2, G, C), jnp.int32),
            pltpu.VMEM((C, D), jnp.float32),
            pltpu.VMEM((C, D), jnp.float32),
            pltpu.VMEM((C, D), jnp.float32),
            pltpu.VMEM((C, D), jnp.float32),
            pltpu.VMEM((C, D), jnp.float32),
            pltpu.VMEM((C, D), jnp.float32),
            pltpu.VMEM_SHARED((NPAD, D), jnp.float32),
            pltpu.SemaphoreType.DMA,
            pltpu.SemaphoreType.DMA,
            pltpu.SemaphoreType.DMA,
            pltpu.SemaphoreType.DMA,
            pltpu.SemaphoreType.DMA,
            pltpu.SemaphoreType.DMA,
            pltpu.SemaphoreType.DMA,
        ],
    )(_sc_body)


def kernel(x, edge_index, edge_attr, u, batch, W1, b1, W2, b2):
    src = edge_index[0].astype(jnp.int32)
    dst = edge_index[1].astype(jnp.int32)
    batch2d = batch.astype(jnp.int32).reshape(N_NODES, 1)

    BE = 16000
    td, ts, eterm = pl.pallas_call(
        _tc_pre_body,
        grid=(N_EDGES // BE,),
        in_specs=[
            pl.BlockSpec((N_NODES, D), lambda i: (0, 0)),
            pl.BlockSpec((IN1, D), lambda i: (0, 0)),
            pl.BlockSpec((1, D), lambda i: (0, 0)),
            pl.BlockSpec((N_GRAPHS, D_U), lambda i: (0, 0)),
            pl.BlockSpec((N_NODES, 1), lambda i: (0, 0)),
            pl.BlockSpec((BE, D_EDGE), lambda i: (i, 0)),
        ],
        out_specs=[
            pl.BlockSpec((N_NODES, D), lambda i: (0, 0)),
            pl.BlockSpec((N_NODES, D), lambda i: (0, 0)),
            pl.BlockSpec((BE // C, C, D), lambda i: (i, 0, 0)),
        ],
        out_shape=[
            jax.ShapeDtypeStruct((N_NODES, D), jnp.float32),
            jax.ShapeDtypeStruct((N_NODES, D), jnp.float32),
            jax.ShapeDtypeStruct((N_EDGES // C, C, D), jnp.float32),
        ],
    )(x, W1, b1.reshape(1, D), u, batch2d, edge_attr)

    agg2 = _sc_call()(td, ts, eterm,
                      dst.reshape(NW, N_GROUPS, G, C),
                      src.reshape(NW, N_GROUPS, G, C))

    out = pl.pallas_call(
        _tc_post_body,
        out_shape=jax.ShapeDtypeStruct((N_NODES, D), jnp.float32),
    )(x, agg2, u, batch2d, W2, b2.reshape(1, D))
    return out
